# baseline (speedup 1.0000x reference)
import functools

import jax
import jax.numpy as jnp
from jax import lax
from jax.experimental import pallas as pl
from jax.experimental.pallas import tpu as pltpu
from jax.experimental.pallas import tpu_sc as plsc


def _sc_gather_wide(H128, idx_flat, N, F):
    TOT = idx_flat.shape[0]
    info = plsc.get_sparse_core_info()
    NC, NS, L = info.num_cores, info.num_subcores, info.num_lanes
    NW = NC * NS
    per_w = TOT // NW
    CHUNK = 128
    n_chunks = per_w // CHUNK
    assert per_w % CHUNK == 0 and per_w % L == 0 and TOT % NW == 0
    assert n_chunks % 2 == 0

    mesh = plsc.VectorSubcoreMesh(core_axis_name="c", subcore_axis_name="s")

    @functools.partial(
        pl.kernel,
        out_type=jax.ShapeDtypeStruct((TOT, 128), jnp.float32),
        mesh=mesh,
        scratch_types=[
            pltpu.VMEM((per_w,), jnp.int32),
            pltpu.VMEM((per_w,), jnp.int32),
            pltpu.VMEM((2, CHUNK, 128), jnp.float32),
            pltpu.SemaphoreType.DMA,
            pltpu.SemaphoreType.DMA,
        ],
    )
    def k(h_hbm, idx_hbm, out_hbm, idxraw_v, q_v, rows_v, sem0, sem1):
        wid = lax.axis_index("s") * NC + lax.axis_index("c")
        base = wid * per_w
        iota = lax.broadcasted_iota(jnp.int32, (L,), 0)
        pltpu.sync_copy(idx_hbm.at[pl.ds(base, per_w)], idxraw_v)

        def compute_rows(t, carry):
            p = base + t * L + iota
            b_of_p = lax.div(p, F)
            r = idxraw_v[pl.ds(t * L, L)] + b_of_p * N
            q_v[pl.ds(t * L, L)] = lax.shift_right_logical(r, 1)
            return carry

        lax.fori_loop(0, per_w // L, compute_rows, 0)

        def start(c, buf, sem):
            idx_slice = q_v.at[pl.ds(c * CHUNK, CHUNK)]
            pltpu.async_copy(h_hbm.at[idx_slice], rows_v.at[buf], sem)

        def wait(c, buf, sem):
            pltpu.make_async_copy(
                h_hbm.at[q_v.at[pl.ds(c * CHUNK, CHUNK)]],
                rows_v.at[buf], sem).wait()

        def drain(c, buf):
            pltpu.sync_copy(rows_v.at[buf],
                            out_hbm.at[pl.ds(base + c * CHUNK, CHUNK)])

        start(0, 0, sem0)

        def pipelined(c2, carry):
            c = c2 * 2
            wait(c, 0, sem0)
            start(c + 1, 1, sem1)
            drain(c, 0)
            wait(c + 1, 1, sem1)

            @pl.when(c2 < n_chunks // 2 - 1)
            def _():
                start(c + 2, 0, sem0)

            drain(c + 1, 1)
            return carry

        lax.fori_loop(0, n_chunks // 2, pipelined, 0)

    return k(H128, idx_flat)


def _tc_mean_t(HT, N):
    Nn, D, B = HT.shape
    Nb = 8

    def body(h_ref, o_ref):
        i = pl.program_id(0)
        s = jnp.sum(h_ref[...], axis=0)

        @pl.when(i == 0)
        def _():
            o_ref[...] = s * (1.0 / N)

        @pl.when(i > 0)
        def _():
            o_ref[...] += s * (1.0 / N)

    return pl.pallas_call(
        body,
        grid=(Nn // Nb,),
        in_specs=[pl.BlockSpec((Nb, D, B), lambda i: (i, 0, 0))],
        out_specs=pl.BlockSpec((D, B), lambda i: (0, 0)),
        out_shape=jax.ShapeDtypeStruct((D, B), jnp.float32),
    )(HT)


def _tc_dense(gw, par, meanv, Wcat, F, D):
    B, FW = gw.shape
    Bb = 512
    dims = (((1,), (0,)), ((), ()))

    def body(g_ref, p_ref, m_ref, w_ref, o_ref):
        li = lax.broadcasted_iota(jnp.int32, (F, FW), 1)
        si = lax.broadcasted_iota(jnp.int32, (F, FW), 0)
        e2 = (lax.div(li, 128) == si).astype(jnp.float32)
        p_exp = lax.dot_general(p_ref[...], e2, dims,
                                preferred_element_type=jnp.float32)
        half = lax.convert_element_type(
            lax.bitwise_and(
                lax.shift_right_logical(
                    lax.broadcasted_iota(jnp.int32, (Bb, FW), 1), 6),
                1),
            jnp.float32)
        g = jnp.where(p_exp == half, jnp.maximum(g_ref[...], 0.0), 0.0)
        acc = lax.dot_general(g, w_ref[0:FW, :], dims,
                              preferred_element_type=jnp.float32)
        acc = acc + lax.dot_general(m_ref[...], w_ref[FW:FW + D, :], dims,
                                    preferred_element_type=jnp.float32)
        o_ref[...] = jnp.maximum(acc, 0.0)

    return pl.pallas_call(
        body,
        grid=(B // Bb,),
        in_specs=[
            pl.BlockSpec((Bb, FW), lambda i: (i, 0)),
            pl.BlockSpec((Bb, F), lambda i: (i, 0)),
            pl.BlockSpec((Bb, D), lambda i: (i, 0)),
            pl.BlockSpec((FW + D, D), lambda i: (0, 0)),
        ],
        out_specs=pl.BlockSpec((Bb, D), lambda i: (i, 0)),
        out_shape=jax.ShapeDtypeStruct((B, D), jnp.float32),
    )(gw, par, meanv, Wcat)


def kernel(H, indice, W):
    B, N, D = H.shape
    F = indice.shape[1]
    idxf = indice.astype(jnp.int32)
    idx_flat = idxf.reshape(B * F)
    par = lax.bitwise_and(idxf, 1).astype(jnp.float32)
    Wr = W[: F * D].reshape(F, D, D)
    Wdup = jnp.concatenate([Wr, Wr], axis=1).reshape(F * 2 * D, D)
    Wcat = jnp.concatenate([Wdup, W[F * D:]], axis=0)

    H128 = H.reshape(B * N * D // 128, 128)
    HT = jnp.transpose(H, (1, 2, 0))
    meanv = _tc_mean_t(HT, N).T
    idx_flat, meanv = lax.optimization_barrier((idx_flat, meanv))
    gw = _sc_gather_wide(H128, idx_flat, N, F)
    out = _tc_dense(gw.reshape(B, F * 128), par, meanv, Wcat, F, D)
    return out[:, None, :]


# speedup vs baseline: 1.6100x; 1.6100x over previous
import functools

import jax
import jax.numpy as jnp
from jax import lax
from jax.experimental import pallas as pl
from jax.experimental.pallas import tpu as pltpu
from jax.experimental.pallas import tpu_sc as plsc


def _sc_gather_elem(HT1, idx_flat, N, F, D, B):
    TOT = idx_flat.shape[0]
    info = plsc.get_sparse_core_info()
    NC, NS, L = info.num_cores, info.num_subcores, info.num_lanes
    NW = NC * NS
    per_w = TOT // NW
    b_per_w = per_w // F
    BCH = 8
    n_chunks = b_per_w // BCH
    PCH = BCH * F
    ECH = PCH * D
    NT = ECH // 128
    assert per_w % L == 0 and PCH % L == 0 and b_per_w % BCH == 0

    mesh = plsc.VectorSubcoreMesh(core_axis_name="c", subcore_axis_name="s")

    @functools.partial(
        pl.kernel,
        out_type=jax.ShapeDtypeStruct((B, F * D), jnp.float32),
        mesh=mesh,
        compiler_params=pltpu.CompilerParams(needs_layout_passes=False),
        scratch_types=[
            pltpu.VMEM((per_w,), jnp.int32),
            pltpu.VMEM((ECH,), jnp.int32),
            pltpu.VMEM((BCH, F * D), jnp.float32),
            pltpu.SemaphoreType.DMA,
        ],
    )
    def k(h_hbm, idx_hbm, out_hbm, idxraw_v, eidx_v, data_v, sem):
        wid = lax.axis_index("s") * NC + lax.axis_index("c")
        base = wid * per_w
        b0 = wid * b_per_w
        iota = lax.broadcasted_iota(jnp.int32, (L,), 0)
        pltpu.sync_copy(idx_hbm.at[pl.ds(base, per_w)], idxraw_v)

        def chunk(cc, carry):
            def gen16(t2, carry2):
                pos = cc * PCH + t2 * L
                n_vec = idxraw_v[pl.ds(pos, L)]
                b_vec = b0 + lax.div(pos + iota, F)
                src0 = (n_vec * (8 * 32768)
                        + lax.shift_right_logical(b_vec, 7) * 1024
                        + lax.bitwise_and(b_vec, 127))
                dst0 = t2 * (L * D) + iota * D

                def dloop(d, carry3):
                    doff = (lax.shift_right_logical(d, 3) * 32768
                            + lax.bitwise_and(d, 7) * 128)
                    plsc.store_scatter(eidx_v, [dst0 + d], src0 + doff)
                    return carry3

                lax.fori_loop(0, D, dloop, 0)
                return carry2

            lax.fori_loop(0, PCH // L, gen16, 0)

            for j in range(NT):
                pltpu.async_copy(
                    h_hbm.at[eidx_v.at[pl.ds(j * 128, 128)]],
                    data_v.at[j // (F * D // 128),
                              pl.ds((j % (F * D // 128)) * 128, 128)],
                    sem)
            pltpu.make_async_copy(out_hbm.at[pl.ds(0, BCH)], data_v,
                                  sem).wait()
            pltpu.sync_copy(
                data_v, out_hbm.at[pl.ds(b0 + cc * BCH, BCH)])
            return carry

        lax.fori_loop(0, n_chunks, chunk, 0)

    return k(HT1, idx_flat)


def _tc_mean_t(HT, N):
    Nn, D, B = HT.shape
    Nb = 8

    def body(h_ref, o_ref):
        i = pl.program_id(0)
        s = jnp.sum(h_ref[...], axis=0)

        @pl.when(i == 0)
        def _():
            o_ref[...] = s * (1.0 / N)

        @pl.when(i > 0)
        def _():
            o_ref[...] += s * (1.0 / N)

    return pl.pallas_call(
        body,
        grid=(Nn // Nb,),
        in_specs=[pl.BlockSpec((Nb, D, B), lambda i: (i, 0, 0))],
        out_specs=pl.BlockSpec((D, B), lambda i: (0, 0)),
        out_shape=jax.ShapeDtypeStruct((D, B), jnp.float32),
    )(HT)


def _tc_dense(g2d, meanv, W):
    B, FD = g2d.shape
    D = meanv.shape[1]

    Bb = 512
    dims = (((1,), (0,)), ((), ()))

    def body(g_ref, m_ref, w_ref, o_ref):
        g = jnp.maximum(g_ref[...], 0.0)
        acc = lax.dot_general(g, w_ref[0:FD, :], dims,
                              preferred_element_type=jnp.float32)
        acc = acc + lax.dot_general(m_ref[...], w_ref[FD:FD + D, :], dims,
                                    preferred_element_type=jnp.float32)
        o_ref[...] = jnp.maximum(acc, 0.0)

    return pl.pallas_call(
        body,
        grid=(B // Bb,),
        in_specs=[
            pl.BlockSpec((Bb, FD), lambda i: (i, 0)),
            pl.BlockSpec((Bb, D), lambda i: (i, 0)),
            pl.BlockSpec((FD + D, D), lambda i: (0, 0)),
        ],
        out_specs=pl.BlockSpec((Bb, D), lambda i: (i, 0)),
        out_shape=jax.ShapeDtypeStruct((B, D), jnp.float32),
    )(g2d, meanv, W)


def kernel(H, indice, W):
    B, N, D = H.shape
    F = indice.shape[1]
    idx_flat = indice.astype(jnp.int32).reshape(B * F)

    HT = jnp.transpose(H, (1, 2, 0))
    meanv = _tc_mean_t(HT, N).T
    Hphys = HT.reshape(N, D // 8, 8, B // 128, 128)
    Hphys = Hphys.transpose(0, 1, 3, 2, 4).reshape(N * D * B)
    gathered = _sc_gather_elem(Hphys, idx_flat, N, F, D, B)
    out = _tc_dense(gathered, meanv, W)
    return out[:, None, :]


# speedup vs baseline: 1.8167x; 1.1284x over previous
import functools

import jax
import jax.numpy as jnp
from jax import lax
from jax.experimental import pallas as pl
from jax.experimental.pallas import tpu as pltpu
from jax.experimental.pallas import tpu_sc as plsc


def _sc_gather_elem(HT1, idx_flat, N, F, D, B):
    TOT = idx_flat.shape[0]
    info = plsc.get_sparse_core_info()
    NC, NS, L = info.num_cores, info.num_subcores, info.num_lanes
    NW = NC * NS
    per_w = TOT // NW
    b_per_w = per_w // F
    BCH = 8
    n_chunks = b_per_w // BCH
    PCH = BCH * F
    ECH = PCH * D
    NT = ECH // 128
    assert per_w % L == 0 and PCH % L == 0 and b_per_w % BCH == 0

    mesh = plsc.VectorSubcoreMesh(core_axis_name="c", subcore_axis_name="s")

    @functools.partial(
        pl.kernel,
        out_type=jax.ShapeDtypeStruct((B, F * D), jnp.float32),
        mesh=mesh,
        compiler_params=pltpu.CompilerParams(needs_layout_passes=False),
        scratch_types=[
            pltpu.VMEM((per_w,), jnp.int32),
            pltpu.VMEM((ECH,), jnp.int32),
            pltpu.VMEM((ECH,), jnp.int32),
            pltpu.VMEM((BCH, F * D), jnp.float32),
            pltpu.VMEM((BCH, F * D), jnp.float32),
            pltpu.SemaphoreType.DMA,
            pltpu.SemaphoreType.DMA,
        ],
    )
    def k(h_hbm, idx_hbm, out_hbm, idxraw_v, eidx0_v, eidx1_v,
          data0_v, data1_v, sem0, sem1):
        eidx_b = (eidx0_v, eidx1_v)
        data_b = (data0_v, data1_v)
        wid = lax.axis_index("s") * NC + lax.axis_index("c")
        base = wid * per_w
        b0 = wid * b_per_w
        iota = lax.broadcasted_iota(jnp.int32, (L,), 0)
        pltpu.sync_copy(idx_hbm.at[pl.ds(base, per_w)], idxraw_v)

        def gen(cc, buf):
            def gen16(t2, carry2):
                pos = cc * PCH + t2 * L
                n_vec = idxraw_v[pl.ds(pos, L)]
                b_vec = b0 + lax.div(pos + iota, F)
                src0 = (n_vec * (8 * 32768)
                        + lax.shift_right_logical(b_vec, 7) * 1024
                        + lax.bitwise_and(b_vec, 127))
                dst0 = t2 * (L * D) + iota * D

                def dloop(dd, carry3):
                    d = dd * 4
                    for kk in range(4):
                        doff = (lax.shift_right_logical(d + kk, 3) * 32768
                                + lax.bitwise_and(d + kk, 7) * 128)
                        plsc.store_scatter(eidx_b[buf],
                                           [dst0 + d + kk], src0 + doff)
                    return carry3

                lax.fori_loop(0, D // 4, dloop, 0)
                return carry2

            lax.fori_loop(0, PCH // L, gen16, 0)

        def fire(buf, sem):
            for j in range(NT):
                pltpu.async_copy(
                    h_hbm.at[eidx_b[buf].at[pl.ds(j * 128, 128)]],
                    data_b[buf].at[j // (F * D // 128),
                                   pl.ds((j % (F * D // 128)) * 128, 128)],
                    sem)

        def drain(buf, sem):
            pltpu.make_async_copy(out_hbm.at[pl.ds(0, BCH)],
                                  data_b[buf], sem).wait()

        def copyout(cc, buf):
            pltpu.sync_copy(data_b[buf],
                            out_hbm.at[pl.ds(b0 + cc * BCH, BCH)])

        gen(0, 0)
        fire(0, sem0)

        def pipelined(cc2, carry):
            c = cc2 * 2
            gen(c + 1, 1)
            fire(1, sem1)
            drain(0, sem0)
            copyout(c, 0)

            @pl.when(cc2 < n_chunks // 2 - 1)
            def _():
                gen(c + 2, 0)
                fire(0, sem0)

            drain(1, sem1)
            copyout(c + 1, 1)
            return carry

        lax.fori_loop(0, n_chunks // 2, pipelined, 0)

    return k(HT1, idx_flat)


def _tc_mean_t(HT, N):
    Nn, D, B = HT.shape
    Nb = 8

    def body(h_ref, o_ref):
        i = pl.program_id(0)
        s = jnp.sum(h_ref[...], axis=0)

        @pl.when(i == 0)
        def _():
            o_ref[...] = s * (1.0 / N)

        @pl.when(i > 0)
        def _():
            o_ref[...] += s * (1.0 / N)

    return pl.pallas_call(
        body,
        grid=(Nn // Nb,),
        in_specs=[pl.BlockSpec((Nb, D, B), lambda i: (i, 0, 0))],
        out_specs=pl.BlockSpec((D, B), lambda i: (0, 0)),
        out_shape=jax.ShapeDtypeStruct((D, B), jnp.float32),
    )(HT)


def _tc_dense(g2d, meanv, W):
    B, FD = g2d.shape
    D = meanv.shape[1]

    Bb = 512
    dims = (((1,), (0,)), ((), ()))

    def body(g_ref, m_ref, w_ref, o_ref):
        g = jnp.maximum(g_ref[...], 0.0)
        acc = lax.dot_general(g, w_ref[0:FD, :], dims,
                              preferred_element_type=jnp.float32)
        acc = acc + lax.dot_general(m_ref[...], w_ref[FD:FD + D, :], dims,
                                    preferred_element_type=jnp.float32)
        o_ref[...] = jnp.maximum(acc, 0.0)

    return pl.pallas_call(
        body,
        grid=(B // Bb,),
        in_specs=[
            pl.BlockSpec((Bb, FD), lambda i: (i, 0)),
            pl.BlockSpec((Bb, D), lambda i: (i, 0)),
            pl.BlockSpec((FD + D, D), lambda i: (0, 0)),
        ],
        out_specs=pl.BlockSpec((Bb, D), lambda i: (i, 0)),
        out_shape=jax.ShapeDtypeStruct((B, D), jnp.float32),
    )(g2d, meanv, W)


def kernel(H, indice, W):
    B, N, D = H.shape
    F = indice.shape[1]
    idx_flat = indice.astype(jnp.int32).reshape(B * F)

    HT = jnp.transpose(H, (1, 2, 0))
    meanv = _tc_mean_t(HT, N).T
    Hphys = HT.reshape(N, D // 8, 8, B // 128, 128)
    Hphys = Hphys.transpose(0, 1, 3, 2, 4).reshape(N * D * B)
    gathered = _sc_gather_elem(Hphys, idx_flat, N, F, D, B)
    out = _tc_dense(gathered, meanv, W)
    return out[:, None, :]


# speedup vs baseline: 2.0271x; 1.1158x over previous
import functools

import jax
import jax.numpy as jnp
from jax import lax
from jax.experimental import pallas as pl
from jax.experimental.pallas import tpu as pltpu
from jax.experimental.pallas import tpu_sc as plsc


def _sc_gather_elem(HT1, idx_flat, N, F, D, B):
    TOT = idx_flat.shape[0]
    info = plsc.get_sparse_core_info()
    NC, NS, L = info.num_cores, info.num_subcores, info.num_lanes
    NW = NC * NS
    per_w = TOT // NW
    b_per_w = per_w // F
    BCH = 8
    n_chunks = b_per_w // BCH
    PCH = BCH * F
    ECH = PCH * D
    NT = ECH // 128
    assert per_w % L == 0 and PCH % L == 0 and b_per_w % BCH == 0

    mesh = plsc.VectorSubcoreMesh(core_axis_name="c", subcore_axis_name="s")

    @functools.partial(
        pl.kernel,
        out_type=jax.ShapeDtypeStruct((B, F * D), jnp.float32),
        mesh=mesh,
        compiler_params=pltpu.CompilerParams(needs_layout_passes=False),
        scratch_types=[
            pltpu.VMEM((per_w,), jnp.int32),
            pltpu.VMEM((ECH,), jnp.int32),
            pltpu.VMEM((ECH,), jnp.int32),
            pltpu.VMEM((BCH, F * D), jnp.float32),
            pltpu.VMEM((BCH, F * D), jnp.float32),
            pltpu.SemaphoreType.DMA,
            pltpu.SemaphoreType.DMA,
            pltpu.SemaphoreType.DMA,
            pltpu.SemaphoreType.DMA,
        ],
    )
    def k(h_hbm, idx_hbm, out_hbm, idxraw_v, eidx0_v, eidx1_v,
          data0_v, data1_v, sem0, sem1, osem0, osem1):
        eidx_b = (eidx0_v, eidx1_v)
        data_b = (data0_v, data1_v)
        wid = lax.axis_index("s") * NC + lax.axis_index("c")
        base = wid * per_w
        b0 = wid * b_per_w
        iota = lax.broadcasted_iota(jnp.int32, (L,), 0)
        pltpu.sync_copy(idx_hbm.at[pl.ds(base, per_w)], idxraw_v)

        def gen(cc, buf):
            def gen16(t2, carry2):
                pos = cc * PCH + t2 * L
                n_vec = idxraw_v[pl.ds(pos, L)]
                b_vec = b0 + lax.div(pos + iota, F)
                src0 = (n_vec * (8 * 32768)
                        + lax.shift_right_logical(b_vec, 7) * 1024
                        + lax.bitwise_and(b_vec, 127))
                dst0 = t2 * (L * D) + iota * D

                def dloop(dd, carry3):
                    d = dd * 4
                    for kk in range(4):
                        doff = (lax.shift_right_logical(d + kk, 3) * 32768
                                + lax.bitwise_and(d + kk, 7) * 128)
                        plsc.store_scatter(eidx_b[buf],
                                           [dst0 + d + kk], src0 + doff)
                    return carry3

                lax.fori_loop(0, D // 4, dloop, 0)
                return carry2

            lax.fori_loop(0, PCH // L, gen16, 0)

        def fire(buf, sem):
            for j in range(NT):
                pltpu.async_copy(
                    h_hbm.at[eidx_b[buf].at[pl.ds(j * 128, 128)]],
                    data_b[buf].at[j // (F * D // 128),
                                   pl.ds((j % (F * D // 128)) * 128, 128)],
                    sem)

        def drain(buf, sem):
            pltpu.make_async_copy(out_hbm.at[pl.ds(0, BCH)],
                                  data_b[buf], sem).wait()

        osem_b = (osem0, osem1)

        def copyout(cc, buf):
            pltpu.async_copy(data_b[buf],
                             out_hbm.at[pl.ds(b0 + cc * BCH, BCH)],
                             osem_b[buf])

        def copyout_wait(buf):
            pltpu.make_async_copy(data_b[buf], out_hbm.at[pl.ds(0, BCH)],
                                  osem_b[buf]).wait()

        gen(0, 0)
        fire(0, sem0)

        def pipelined(cc2, carry):
            c = cc2 * 2
            gen(c + 1, 1)

            @pl.when(cc2 > 0)
            def _():
                copyout_wait(1)

            fire(1, sem1)
            drain(0, sem0)
            copyout(c, 0)

            @pl.when(cc2 < n_chunks // 2 - 1)
            def _():
                gen(c + 2, 0)
                copyout_wait(0)
                fire(0, sem0)

            drain(1, sem1)
            copyout(c + 1, 1)
            return carry

        lax.fori_loop(0, n_chunks // 2, pipelined, 0)
        copyout_wait(0)
        copyout_wait(1)

    return k(HT1, idx_flat)


def _tc_mean_t(HT, N):
    Nn, D, B = HT.shape
    Nb = 8

    def body(h_ref, o_ref):
        i = pl.program_id(0)
        s = jnp.sum(h_ref[...], axis=0)

        @pl.when(i == 0)
        def _():
            o_ref[...] = s * (1.0 / N)

        @pl.when(i > 0)
        def _():
            o_ref[...] += s * (1.0 / N)

    return pl.pallas_call(
        body,
        grid=(Nn // Nb,),
        in_specs=[pl.BlockSpec((Nb, D, B), lambda i: (i, 0, 0))],
        out_specs=pl.BlockSpec((D, B), lambda i: (0, 0)),
        out_shape=jax.ShapeDtypeStruct((D, B), jnp.float32),
    )(HT)


def _tc_dense(g2d, meanv, W):
    B, FD = g2d.shape
    D = meanv.shape[1]

    Bb = 512
    dims = (((1,), (0,)), ((), ()))

    def body(g_ref, m_ref, w_ref, o_ref):
        g = jnp.maximum(g_ref[...], 0.0)
        acc = lax.dot_general(g, w_ref[0:FD, :], dims,
                              preferred_element_type=jnp.float32)
        acc = acc + lax.dot_general(m_ref[...], w_ref[FD:FD + D, :], dims,
                                    preferred_element_type=jnp.float32)
        o_ref[...] = jnp.maximum(acc, 0.0)

    return pl.pallas_call(
        body,
        grid=(B // Bb,),
        in_specs=[
            pl.BlockSpec((Bb, FD), lambda i: (i, 0)),
            pl.BlockSpec((Bb, D), lambda i: (i, 0)),
            pl.BlockSpec((FD + D, D), lambda i: (0, 0)),
        ],
        out_specs=pl.BlockSpec((Bb, D), lambda i: (i, 0)),
        out_shape=jax.ShapeDtypeStruct((B, D), jnp.float32),
    )(g2d, meanv, W)


def kernel(H, indice, W):
    B, N, D = H.shape
    F = indice.shape[1]
    idx_flat = indice.astype(jnp.int32).reshape(B * F)

    HT = jnp.transpose(H, (1, 2, 0))
    meanv = _tc_mean_t(HT, N).T
    Hphys = HT.reshape(N, D // 8, 8, B // 128, 128)
    Hphys = Hphys.transpose(0, 1, 3, 2, 4).reshape(N * D * B)
    gathered = _sc_gather_elem(Hphys, idx_flat, N, F, D, B)
    out = _tc_dense(gathered, meanv, W)
    return out[:, None, :]
